# baseline (speedup 1.0000x reference)
import functools

import numpy as np
import jax
import jax.numpy as jnp
from jax import lax
from jax.experimental import pallas as pl
from jax.experimental.pallas import tpu as pltpu
from jax.experimental.pallas import tpu_sc as plsc

N = 10000
E = 320000
C = 128
NC = 2
NS = 16
NW = NC * NS
EW = E // NW
CH = 80
T = EW // CH
NB = 3
NG = (T + NB - 1) // NB
RPW = 624
TAIL = N - NS * RPW
ZR = 24
NZ = RPW // ZR

PERM = np.concatenate([
    np.concatenate([g * 32 + 2 * np.arange(16), g * 32 + 2 * np.arange(16) + 1])
    for g in range(C // 32)
])


def _sc_aggregate(xbf, ecol, eval_, erow):
    mesh = plsc.VectorSubcoreMesh(core_axis_name="c", subcore_axis_name="s")

    @functools.partial(
        pl.kernel,
        out_type=jax.ShapeDtypeStruct((NC, N, C), jnp.float32),
        mesh=mesh,
        compiler_params=pltpu.CompilerParams(
            needs_layout_passes=False, use_tc_tiling_on_sc=False),
        scratch_types=[
            pltpu.VMEM_SHARED((N, C), jnp.float32),
            [pltpu.VMEM((CH, C // 2), jnp.int32) for _ in range(NB)],
            [pltpu.VMEM((CH, C), jnp.float32) for _ in range(NB)],
            [pltpu.VMEM((1, CH), jnp.int32) for _ in range(NB)],
            [pltpu.VMEM((1, CH), jnp.float32) for _ in range(NB)],
            [pltpu.VMEM((1, CH), jnp.int32) for _ in range(NB)],
            [pltpu.VMEM((1, CH), jnp.int32) for _ in range(NB)],
            pltpu.VMEM((ZR, C), jnp.float32),
            [pltpu.SemaphoreType.DMA for _ in range(NB)],
            [pltpu.SemaphoreType.DMA for _ in range(NB)],
            [pltpu.SemaphoreType.DMA for _ in range(NB)],
            [pltpu.SemaphoreType.DMA for _ in range(NB)],
            [pltpu.SemaphoreType.DMA for _ in range(NB)],
        ],
    )
    def agg(xbf_hbm, ecol_hbm, eval_hbm, erow_hbm, out_hbm,
            acc, rbf, rfl, cb, vb, rb, sb, zbuf,
            gsem, ssem, csem, vsem, rsem):
        cid = lax.axis_index("c")
        sid = lax.axis_index("s")
        wid = sid * NC + cid

        zero16 = jnp.zeros((16,), jnp.float32)

        @pl.loop(0, ZR)
        def _(i):
            for j in range(C // 16):
                zbuf[i, pl.ds(j * 16, 16)] = zero16

        @pl.loop(0, NZ)
        def _(k):
            pltpu.sync_copy(zbuf, acc.at[pl.ds(sid * RPW + k * ZR, ZR)])

        @pl.when(sid == NS - 1)
        def _():
            pltpu.sync_copy(zbuf.at[pl.ds(0, TAIL)],
                            acc.at[pl.ds(NS * RPW, TAIL)])

        plsc.subcore_barrier()

        def issue_cv(t, b):
            pltpu.async_copy(ecol_hbm.at[wid, t], cb[b], csem[b])
            pltpu.async_copy(eval_hbm.at[wid, t], vb[b], vsem[b])

        def issue_r(t, b):
            pltpu.async_copy(erow_hbm.at[wid, t], rb[b], rsem[b])

        def issue_gather(t, b):
            pltpu.make_async_copy(ecol_hbm.at[0, 0], cb[b], csem[b]).wait()
            pltpu.async_copy(xbf_hbm.at[cb[b].at[0]], rbf[b], gsem[b])

        for b in range(NB):
            issue_cv(b, b)
            issue_r(b, b)
        issue_gather(0, 0)
        issue_gather(1, 1)

        @pl.loop(0, NG)
        def _(g):
            for u in range(NB):
                b = u
                b2 = (u + 2) % NB
                t = g * NB + u

                @pl.when(t < T)
                def _():
                    pltpu.make_async_copy(
                        xbf_hbm.at[pl.ds(0, CH)], rbf[b], gsem[b]).wait()

                    @pl.when(t + 2 < T)
                    def _():
                        pltpu.make_async_copy(
                            ecol_hbm.at[0, 0], cb[b2], csem[b2]).wait()
                        pltpu.async_copy(
                            xbf_hbm.at[cb[b2].at[0]], rbf[b2], gsem[b2])

                    @pl.when(t >= NB)
                    def _():
                        pltpu.make_async_copy(
                            out_hbm.at[0, pl.ds(0, CH)], rfl[b],
                            ssem[b]).wait()
                    pltpu.make_async_copy(
                        eval_hbm.at[0, 0], vb[b], vsem[b]).wait()
                    himask = jnp.full((16,), -65536, jnp.int32)
                    for grp in range(CH // 16):
                        vg = vb[b][0, pl.ds(grp * 16, 16)]
                        for l in range(16):
                            v = vg[l]
                            e = grp * 16 + l
                            for h in range(C // 32):
                                w = rbf[b][e, pl.ds(h * 16, 16)]
                                lo = plsc.bitcast(w << 16, jnp.float32)
                                hi = plsc.bitcast(w & himask, jnp.float32)
                                rfl[b][e, pl.ds(h * 32, 16)] = lo * v
                                rfl[b][e, pl.ds(h * 32 + 16, 16)] = hi * v
                    pltpu.make_async_copy(
                        erow_hbm.at[0, 0], rb[b], rsem[b]).wait()
                    for q in range(CH // 16):
                        sb[b][0, pl.ds(q * 16, 16)] = rb[b][0, pl.ds(q * 16, 16)]
                    pltpu.async_copy(
                        rfl[b], acc.at[sb[b].at[0]], ssem[b], add=True)

                @pl.when(t + NB < T)
                def _():
                    issue_cv(t + NB, b)
                    issue_r(t + NB, b)


        for b in range(NB):
            pltpu.make_async_copy(
                out_hbm.at[0, pl.ds(0, CH)], rfl[b], ssem[b]).wait()

        plsc.subcore_barrier()

        @pl.loop(0, NZ)
        def _(k):
            start = sid * RPW + k * ZR
            pltpu.sync_copy(acc.at[pl.ds(start, ZR)], zbuf)
            pltpu.sync_copy(zbuf, out_hbm.at[cid, pl.ds(start, ZR)])

        @pl.when(sid == NS - 1)
        def _():
            pltpu.sync_copy(acc.at[pl.ds(NS * RPW, TAIL)],
                            zbuf.at[pl.ds(0, TAIL)])
            pltpu.sync_copy(zbuf.at[pl.ds(0, TAIL)],
                            out_hbm.at[cid, pl.ds(NS * RPW, TAIL)])

    return agg(xbf, ecol, eval_, erow)


def _tc_linear(partials, Wp, b2d):
    BLK = 1000

    def body(p_ref, w_ref, b_ref, o_ref):
        y = p_ref[0] + p_ref[1]
        o_ref[...] = lax.dot_general(
            y, w_ref[...], (((1,), (1,)), ((), ())),
            preferred_element_type=jnp.float32) + b_ref[...]

    return pl.pallas_call(
        body,
        grid=(N // BLK,),
        in_specs=[
            pl.BlockSpec((NC, BLK, C), lambda i: (0, i, 0)),
            pl.BlockSpec((C, C), lambda i: (0, 0)),
            pl.BlockSpec((1, C), lambda i: (0, 0)),
        ],
        out_specs=pl.BlockSpec((BLK, C), lambda i: (i, 0)),
        out_shape=jax.ShapeDtypeStruct((N, C), jnp.float32),
    )(partials, Wp, b2d)


@jax.jit
def kernel(x, adj_indices, adj_values, W, b):
    x2d = x.reshape(N, C)
    xbf = lax.bitcast_convert_type(
        x2d.astype(jnp.bfloat16).reshape(N, C // 2, 2), jnp.int32)
    col = adj_indices[1].astype(jnp.int32)
    row = adj_indices[0].astype(jnp.int32)
    ecol = col.reshape(NW, T, 1, CH)
    eval_ = adj_values.reshape(NW, T, 1, CH)
    erow = row.reshape(NW, T, 1, CH)
    partials = _sc_aggregate(xbf, ecol, eval_, erow)
    Wp = W[:, PERM]
    out = _tc_linear(partials, Wp, b.reshape(1, C))
    return out.reshape(1, N, C)


# speedup vs baseline: 1.2583x; 1.2583x over previous
import functools

import jax
import jax.numpy as jnp
from jax import lax
from jax.experimental import pallas as pl
from jax.experimental.pallas import tpu as pltpu
from jax.experimental.pallas import tpu_sc as plsc

N = 10000
E = 320000
C = 128
NC = 2
NS = 16
NW = NC * NS
EW = E // NW
CH = 80
T = EW // CH
NB = 4
NG = (T + NB - 1) // NB
RPW = 624
TAIL = N - NS * RPW
ZR = 48
NZ = RPW // ZR


def _sc_aggregate(x, ecol, eval_, erow):
    mesh = plsc.VectorSubcoreMesh(core_axis_name="c", subcore_axis_name="s")

    @functools.partial(
        pl.kernel,
        out_type=jax.ShapeDtypeStruct((NC, N, C), jnp.float32),
        mesh=mesh,
        scratch_types=[
            pltpu.VMEM_SHARED((N, C), jnp.float32),
            [pltpu.VMEM((1, CH), jnp.int32) for _ in range(NB)],
            [pltpu.VMEM((1, CH), jnp.float32) for _ in range(NB)],
            [pltpu.VMEM((1, CH), jnp.int32) for _ in range(NB)],
            [pltpu.VMEM((CH, C), jnp.float32) for _ in range(NB)],
            pltpu.VMEM((ZR, C), jnp.float32),
            [pltpu.SemaphoreType.DMA for _ in range(NB)],
            [pltpu.SemaphoreType.DMA for _ in range(NB)],
            [pltpu.SemaphoreType.DMA for _ in range(NB)],
            [pltpu.SemaphoreType.DMA for _ in range(NB)],
            [pltpu.SemaphoreType.DMA for _ in range(NB)],
        ],
    )
    def agg(x_hbm, ecol_hbm, eval_hbm, erow_hbm, out_hbm,
            acc, cb, vb, rb, rows, zbuf, gsem, ssem, csem, vsem, rsem):
        cid = lax.axis_index("c")
        sid = lax.axis_index("s")
        wid = sid * NC + cid

        zero16 = jnp.zeros((16,), jnp.float32)

        @pl.loop(0, ZR)
        def _(i):
            for j in range(C // 16):
                zbuf[i, pl.ds(j * 16, 16)] = zero16

        @pl.loop(0, NZ)
        def _(k):
            pltpu.sync_copy(zbuf, acc.at[pl.ds(sid * RPW + k * ZR, ZR)])

        @pl.when(sid == NS - 1)
        def _():
            pltpu.sync_copy(zbuf.at[pl.ds(0, TAIL)],
                            acc.at[pl.ds(NS * RPW, TAIL)])

        plsc.subcore_barrier()

        def issue_cv(t, b):
            pltpu.async_copy(ecol_hbm.at[wid, t], cb[b], csem[b])
            pltpu.async_copy(eval_hbm.at[wid, t], vb[b], vsem[b])

        def issue_r(t, b):
            pltpu.async_copy(erow_hbm.at[wid, t], rb[b], rsem[b])

        def issue_gather(t, b):
            pltpu.make_async_copy(ecol_hbm.at[0, 0], cb[b], csem[b]).wait()
            pltpu.async_copy(x_hbm.at[cb[b].at[0]], rows[b], gsem[b])

        for b in range(NB):
            issue_cv(b, b)
        for b in range(3):
            issue_r(b, b)
            issue_gather(b, b)

        @pl.loop(0, NG)
        def _(g):
            for u in range(NB):
                b = u
                t = g * NB + u
                live = t < T

                @pl.when(live)
                def _():
                    pltpu.make_async_copy(
                        x_hbm.at[pl.ds(0, CH)], rows[b], gsem[b]).wait()
                    pltpu.make_async_copy(
                        eval_hbm.at[0, 0], vb[b], vsem[b]).wait()
                    for grp in range(CH // 16):
                        vg = vb[b][0, pl.ds(grp * 16, 16)]
                        for l in range(16):
                            v = vg[l]
                            e = grp * 16 + l
                            for j in range(C // 16):
                                sl = pl.ds(j * 16, 16)
                                rows[b][e, sl] = rows[b][e, sl] * v
                    pltpu.make_async_copy(
                        erow_hbm.at[0, 0], rb[b], rsem[b]).wait()
                    pltpu.async_copy(
                        rows[b], acc.at[rb[b].at[0]], ssem[b], add=True)

                b2 = (u + 3) % NB

                @pl.when(t + 3 < T)
                def _():
                    @pl.when(t >= 1)
                    def _():
                        pltpu.make_async_copy(
                            x_hbm.at[pl.ds(0, CH)], rows[b2], ssem[b2]).wait()
                    issue_r(t + 3, b2)
                    pltpu.make_async_copy(
                        ecol_hbm.at[0, 0], cb[b2], csem[b2]).wait()
                    pltpu.async_copy(
                        x_hbm.at[cb[b2].at[0]], rows[b2], gsem[b2])

                @pl.when(t + 4 < T)
                def _():
                    issue_cv(t + 4, b)

        for b in range(NB):
            pltpu.make_async_copy(
                x_hbm.at[pl.ds(0, CH)], rows[b], ssem[b]).wait()

        plsc.subcore_barrier()

        @pl.loop(0, NZ)
        def _(k):
            start = sid * RPW + k * ZR
            pltpu.sync_copy(acc.at[pl.ds(start, ZR)], zbuf)
            pltpu.sync_copy(zbuf, out_hbm.at[cid, pl.ds(start, ZR)])

        @pl.when(sid == NS - 1)
        def _():
            pltpu.sync_copy(acc.at[pl.ds(NS * RPW, TAIL)],
                            zbuf.at[pl.ds(0, TAIL)])
            pltpu.sync_copy(zbuf.at[pl.ds(0, TAIL)],
                            out_hbm.at[cid, pl.ds(NS * RPW, TAIL)])

    return agg(x, ecol, eval_, erow)


def _tc_linear(partials, W, b2d):
    BLK = 1000

    def body(p_ref, w_ref, b_ref, o_ref):
        y = p_ref[0] + p_ref[1]
        o_ref[...] = lax.dot_general(
            y, w_ref[...], (((1,), (1,)), ((), ())),
            preferred_element_type=jnp.float32) + b_ref[...]

    return pl.pallas_call(
        body,
        grid=(N // BLK,),
        in_specs=[
            pl.BlockSpec((NC, BLK, C), lambda i: (0, i, 0)),
            pl.BlockSpec((C, C), lambda i: (0, 0)),
            pl.BlockSpec((1, C), lambda i: (0, 0)),
        ],
        out_specs=pl.BlockSpec((BLK, C), lambda i: (i, 0)),
        out_shape=jax.ShapeDtypeStruct((N, C), jnp.float32),
    )(partials, W, b2d)


@jax.jit
def kernel(x, adj_indices, adj_values, W, b):
    x2d = x.reshape(N, C)
    col = adj_indices[1].astype(jnp.int32)
    row = adj_indices[0].astype(jnp.int32)
    ecol = col.reshape(NW, T, 1, CH)
    eval_ = adj_values.reshape(NW, T, 1, CH)
    erow = row.reshape(NW, T, 1, CH)
    partials = _sc_aggregate(x2d, ecol, eval_, erow)
    out = _tc_linear(partials, W, b.reshape(1, C))
    return out.reshape(1, N, C)
